# baseline (speedup 1.0000x reference)
import functools

import jax
import jax.numpy as jnp
from jax import lax
from jax.experimental import pallas as pl
from jax.experimental.pallas import tpu as pltpu
from jax.experimental.pallas import tpu_sc as plsc


def _make_lookup(B, V, D):
  info = plsc.get_sparse_core_info()
  NC, NS = info.num_cores, info.num_subcores
  NW = NC * NS
  b_per_w = B // NW
  CH = 128
  n_ch = b_per_w // CH

  mesh = plsc.VectorSubcoreMesh(core_axis_name="c", subcore_axis_name="s")

  @functools.partial(
      pl.kernel,
      mesh=mesh,
      out_type=jax.ShapeDtypeStruct((B, D), jnp.float32),
      scratch_types=[
          pltpu.VMEM((n_ch, CH), jnp.int32),
          pltpu.VMEM((b_per_w, D), jnp.float32),
          pltpu.SemaphoreType.DMA((n_ch,)),
          pltpu.SemaphoreType.DMA,
      ],
  )
  def lookup(t_hbm, table_hbm, out_hbm, idx_v, rows_v, gsem, ssem):
    wid = lax.axis_index("s") * NC + lax.axis_index("c")
    base = wid * b_per_w
    pltpu.sync_copy(t_hbm.at[wid], idx_v)
    gathers = []
    for j in range(n_ch):
      gathers.append(
          pltpu.async_copy(
              table_hbm.at[idx_v.at[j]],
              rows_v.at[pl.ds(j * CH, CH)],
              gsem.at[j],
          ))
    stores = []
    for j in range(n_ch):
      gathers[j].wait()
      stores.append(
          pltpu.async_copy(
              rows_v.at[pl.ds(j * CH, CH)],
              out_hbm.at[pl.ds(base + j * CH, CH)],
              ssem,
          ))
    for c in stores:
      c.wait()

  return lookup, NW, n_ch, CH


def kernel(t, table):
  B, = t.shape
  V, D = table.shape
  lookup, NW, n_ch, CH = _make_lookup(B, V, D)
  t3 = t.astype(jnp.int32).reshape(NW, n_ch, CH)
  return lookup(t3, table)


# speedup vs baseline: 1.0208x; 1.0208x over previous
import functools

import jax
import jax.numpy as jnp
from jax import lax
from jax.experimental import pallas as pl
from jax.experimental.pallas import tpu as pltpu
from jax.experimental.pallas import tpu_sc as plsc


def _make_lookup(B, V, D):
  info = plsc.get_sparse_core_info()
  NC, NS = info.num_cores, info.num_subcores
  NW = NC * NS
  b_per_w = B // NW
  CH = 128
  n_ch = b_per_w // CH

  mesh = plsc.VectorSubcoreMesh(core_axis_name="c", subcore_axis_name="s")

  @functools.partial(
      pl.kernel,
      mesh=mesh,
      out_type=jax.ShapeDtypeStruct((NW, b_per_w, D), jnp.float32),
      scratch_types=[
          pltpu.VMEM((b_per_w,), jnp.int32),
          pltpu.VMEM((b_per_w, D), jnp.float32),
          pltpu.SemaphoreType.DMA,
      ],
  )
  def lookup(t_hbm, table_hbm, out_hbm, idx_v, rows_v, sem):
    wid = lax.axis_index("s") * NC + lax.axis_index("c")
    pltpu.sync_copy(t_hbm.at[wid], idx_v)
    pltpu.async_copy(table_hbm.at[idx_v], rows_v, sem).wait()
    pltpu.sync_copy(rows_v, out_hbm.at[wid])

  return lookup, NW, n_ch, CH


def kernel(t, table):
  B, = t.shape
  V, D = table.shape
  lookup, NW, n_ch, CH = _make_lookup(B, V, D)
  t3 = t.astype(jnp.int32).reshape(NW, B // NW)
  return lookup(t3, table).reshape(B, D)


# speedup vs baseline: 1.0477x; 1.0264x over previous
import functools

import jax
import jax.numpy as jnp
from jax import lax
from jax.experimental import pallas as pl
from jax.experimental.pallas import tpu as pltpu
from jax.experimental.pallas import tpu_sc as plsc


def _make_lookup(B, V, D):
  info = plsc.get_sparse_core_info()
  NC, NS = info.num_cores, info.num_subcores
  N0 = 456
  N1 = B // NS - N0
  split = N0 * NS

  mesh = plsc.VectorSubcoreMesh(core_axis_name="c", subcore_axis_name="s")

  @functools.partial(
      pl.kernel,
      mesh=mesh,
      out_type=jax.ShapeDtypeStruct((B, D), jnp.float32),
      scratch_types=[
          pltpu.VMEM((max(N0, N1),), jnp.int32),
          pltpu.VMEM((max(N0, N1), D), jnp.float32),
          pltpu.SemaphoreType.DMA,
      ],
  )
  def lookup(t_hbm, table_hbm, out_hbm, idx_v, rows_v, sem):
    c = lax.axis_index("c")
    s = lax.axis_index("s")

    @pl.when(c == 0)
    def _():
      base = s * N0
      pltpu.sync_copy(t_hbm.at[pl.ds(base, N0)], idx_v.at[pl.ds(0, N0)])
      pltpu.async_copy(
          table_hbm.at[idx_v.at[pl.ds(0, N0)]],
          rows_v.at[pl.ds(0, N0)], sem).wait()
      pltpu.sync_copy(rows_v.at[pl.ds(0, N0)], out_hbm.at[pl.ds(base, N0)])

    @pl.when(c == 1)
    def _():
      base = split + s * N1
      pltpu.sync_copy(t_hbm.at[pl.ds(base, N1)], idx_v.at[pl.ds(0, N1)])
      pltpu.async_copy(
          table_hbm.at[idx_v.at[pl.ds(0, N1)]],
          rows_v.at[pl.ds(0, N1)], sem).wait()
      pltpu.sync_copy(rows_v.at[pl.ds(0, N1)], out_hbm.at[pl.ds(base, N1)])

  return lookup


def kernel(t, table):
  B, = t.shape
  V, D = table.shape
  lookup = _make_lookup(B, V, D)
  return lookup(t.astype(jnp.int32), table)


# speedup vs baseline: 1.0662x; 1.0176x over previous
import functools

import jax
import jax.numpy as jnp
from jax import lax
from jax.experimental import pallas as pl
from jax.experimental.pallas import tpu as pltpu
from jax.experimental.pallas import tpu_sc as plsc


def _make_lookup(B, V, D):
  info = plsc.get_sparse_core_info()
  NC, NS = info.num_cores, info.num_subcores
  N0 = 488
  N1 = B // NS - N0
  split = N0 * NS

  mesh = plsc.VectorSubcoreMesh(core_axis_name="c", subcore_axis_name="s")

  @functools.partial(
      pl.kernel,
      mesh=mesh,
      out_type=jax.ShapeDtypeStruct((B, D), jnp.float32),
      scratch_types=[
          pltpu.VMEM((max(N0, N1),), jnp.int32),
          pltpu.VMEM((max(N0, N1), D), jnp.float32),
          pltpu.SemaphoreType.DMA,
      ],
  )
  def lookup(t_hbm, table_hbm, out_hbm, idx_v, rows_v, sem):
    c = lax.axis_index("c")
    s = lax.axis_index("s")

    @pl.when(c == 0)
    def _():
      base = s * N0
      pltpu.sync_copy(t_hbm.at[pl.ds(base, N0)], idx_v.at[pl.ds(0, N0)])
      pltpu.async_copy(
          table_hbm.at[idx_v.at[pl.ds(0, N0)]],
          rows_v.at[pl.ds(0, N0)], sem).wait()
      pltpu.sync_copy(rows_v.at[pl.ds(0, N0)], out_hbm.at[pl.ds(base, N0)])

    @pl.when(c == 1)
    def _():
      base = split + s * N1
      pltpu.sync_copy(t_hbm.at[pl.ds(base, N1)], idx_v.at[pl.ds(0, N1)])
      pltpu.async_copy(
          table_hbm.at[idx_v.at[pl.ds(0, N1)]],
          rows_v.at[pl.ds(0, N1)], sem).wait()
      pltpu.sync_copy(rows_v.at[pl.ds(0, N1)], out_hbm.at[pl.ds(base, N1)])

  return lookup


def kernel(t, table):
  B, = t.shape
  V, D = table.shape
  lookup = _make_lookup(B, V, D)
  return lookup(t.astype(jnp.int32), table)


# speedup vs baseline: 1.0706x; 1.0041x over previous
import functools

import jax
import jax.numpy as jnp
from jax import lax
from jax.experimental import pallas as pl
from jax.experimental.pallas import tpu as pltpu
from jax.experimental.pallas import tpu_sc as plsc


def _make_lookup(B, V, D):
  info = plsc.get_sparse_core_info()
  NC, NS = info.num_cores, info.num_subcores
  N0 = 472
  N1 = B // NS - N0
  split = N0 * NS

  mesh = plsc.VectorSubcoreMesh(core_axis_name="c", subcore_axis_name="s")

  @functools.partial(
      pl.kernel,
      mesh=mesh,
      out_type=jax.ShapeDtypeStruct((B, D), jnp.float32),
      scratch_types=[
          pltpu.VMEM((max(N0, N1),), jnp.int32),
          pltpu.VMEM((max(N0, N1), D), jnp.float32),
          pltpu.SemaphoreType.DMA,
      ],
  )
  def lookup(t_hbm, table_hbm, out_hbm, idx_v, rows_v, sem):
    c = lax.axis_index("c")
    s = lax.axis_index("s")

    @pl.when(c == 0)
    def _():
      base = s * N0
      pltpu.sync_copy(t_hbm.at[pl.ds(base, N0)], idx_v.at[pl.ds(0, N0)])
      pltpu.async_copy(
          table_hbm.at[idx_v.at[pl.ds(0, N0)]],
          rows_v.at[pl.ds(0, N0)], sem).wait()
      pltpu.sync_copy(rows_v.at[pl.ds(0, N0)], out_hbm.at[pl.ds(base, N0)])

    @pl.when(c == 1)
    def _():
      base = split + s * N1
      pltpu.sync_copy(t_hbm.at[pl.ds(base, N1)], idx_v.at[pl.ds(0, N1)])
      pltpu.async_copy(
          table_hbm.at[idx_v.at[pl.ds(0, N1)]],
          rows_v.at[pl.ds(0, N1)], sem).wait()
      pltpu.sync_copy(rows_v.at[pl.ds(0, N1)], out_hbm.at[pl.ds(base, N1)])

  return lookup


def kernel(t, table):
  B, = t.shape
  V, D = table.shape
  lookup = _make_lookup(B, V, D)
  return lookup(t.astype(jnp.int32), table)
